# baseline (speedup 1.0000x reference)
import functools

import jax
import jax.numpy as jnp
from jax import lax
from jax.experimental import pallas as pl
from jax.experimental.pallas import tpu as pltpu
from jax.experimental.pallas import tpu_sc as plsc

_D = 300
_DP = 320
_B = 4096 * 50
_NC = 2
_NS = 16
_NW = _NC * _NS
_BPW = _B // _NW
_CHUNK = 128
_NCHUNK = _BPW // _CHUNK

_mesh = plsc.VectorSubcoreMesh(core_axis_name="c", subcore_axis_name="s")


@functools.partial(
    pl.kernel,
    mesh=_mesh,
    compiler_params=pltpu.CompilerParams(use_tc_tiling_on_sc=False),
    out_type=jax.ShapeDtypeStruct((_B, _D), jnp.float32),
    scratch_types=[
        pltpu.VMEM((_NCHUNK, _CHUNK), jnp.int32),
        pltpu.VMEM((_CHUNK, _DP), jnp.float32),
        pltpu.VMEM((_CHUNK, _D), jnp.float32),
        pltpu.SemaphoreType.DMA,
    ],
)
def _emb_gather(idx_hbm, table_hbm, out_hbm, idx_v, pad_v, cmp_v, sem):
    wid = lax.axis_index("s") * _NC + lax.axis_index("c")
    base = wid * _BPW
    pltpu.sync_copy(idx_hbm.at[wid], idx_v)

    def compact_row(r, carry):
        for k in range(18):
            cmp_v[r, pl.ds(16 * k, 16)] = pad_v[r, pl.ds(16 * k, 16)]
        cmp_v[r, pl.ds(_D - 16, 16)] = pad_v[r, pl.ds(_D - 16, 16)]
        return carry

    def step(j, carry):
        pltpu.async_copy(table_hbm.at[idx_v.at[j]], pad_v, sem).wait()
        lax.fori_loop(0, _CHUNK, compact_row, 0)
        pltpu.sync_copy(cmp_v, out_hbm.at[pl.ds(base + j * _CHUNK, _CHUNK)])
        return carry

    lax.fori_loop(0, _NCHUNK, step, 0)


def kernel(sentence, W):
    idx = sentence.reshape(_NW, _NCHUNK, _CHUNK)
    Wp = jnp.pad(W, ((0, 0), (0, _DP - _D)))
    out = _emb_gather(idx, Wp)
    return out.reshape(sentence.shape[0], sentence.shape[1], _D)


# speedup vs baseline: 1.1675x; 1.1675x over previous
import functools

import jax
import jax.numpy as jnp
from jax import lax
from jax.experimental import pallas as pl
from jax.experimental.pallas import tpu as pltpu
from jax.experimental.pallas import tpu_sc as plsc

_D = 300
_DP = 320
_B = 4096 * 50
_NC = 2
_NS = 16
_NW = _NC * _NS
_BPW = _B // _NW
_CHUNK = 64
_NCHUNK = _BPW // _CHUNK

_mesh = plsc.VectorSubcoreMesh(core_axis_name="c", subcore_axis_name="s")


@functools.partial(
    pl.kernel,
    mesh=_mesh,
    compiler_params=pltpu.CompilerParams(use_tc_tiling_on_sc=False),
    out_type=jax.ShapeDtypeStruct((_B, _D), jnp.float32),
    scratch_types=[
        pltpu.VMEM((_NCHUNK, _CHUNK), jnp.int32),
        pltpu.VMEM((_CHUNK, _DP), jnp.float32),
        pltpu.VMEM((_CHUNK, _DP), jnp.float32),
        pltpu.VMEM((_CHUNK, _D), jnp.float32),
        pltpu.VMEM((_CHUNK, _D), jnp.float32),
        pltpu.SemaphoreType.DMA,
        pltpu.SemaphoreType.DMA,
        pltpu.SemaphoreType.DMA,
        pltpu.SemaphoreType.DMA,
    ],
)
def _emb_gather(idx_hbm, table_hbm, out_hbm, idx_v,
                pad0, pad1, cmp0, cmp1, gs0, gs1, ws0, ws1):
    wid = lax.axis_index("s") * _NC + lax.axis_index("c")
    base = wid * _BPW
    pltpu.sync_copy(idx_hbm.at[wid], idx_v)
    pltpu.async_copy(table_hbm.at[idx_v.at[0]], pad0, gs0)

    def compact(pad_v, cmp_v):
        def rowpair(i, carry):
            for rr in range(2):
                r = 2 * i + rr
                for k in range(18):
                    cmp_v[r, pl.ds(16 * k, 16)] = pad_v[r, pl.ds(16 * k, 16)]
                cmp_v[r, pl.ds(_D - 16, 16)] = pad_v[r, pl.ds(_D - 16, 16)]
            return carry
        lax.fori_loop(0, _CHUNK // 2, rowpair, 0)

    def out_rows(j):
        return out_hbm.at[pl.ds(base + j * _CHUNK, _CHUNK)]

    def pair(t, carry):
        j0 = 2 * t
        pltpu.async_copy(table_hbm.at[idx_v.at[j0 + 1]], pad1, gs1)
        pltpu.make_async_copy(table_hbm.at[idx_v.at[j0]], pad0, gs0).wait()

        @pl.when(t >= 1)
        def _():
            pltpu.make_async_copy(cmp0, out_rows(j0 - 2), ws0).wait()

        compact(pad0, cmp0)
        pltpu.async_copy(cmp0, out_rows(j0), ws0)

        @pl.when(t < _NCHUNK // 2 - 1)
        def _():
            pltpu.async_copy(table_hbm.at[idx_v.at[j0 + 2]], pad0, gs0)

        pltpu.make_async_copy(table_hbm.at[idx_v.at[j0 + 1]], pad1, gs1).wait()

        @pl.when(t >= 1)
        def _():
            pltpu.make_async_copy(cmp1, out_rows(j0 - 1), ws1).wait()

        compact(pad1, cmp1)
        pltpu.async_copy(cmp1, out_rows(j0 + 1), ws1)
        return carry

    lax.fori_loop(0, _NCHUNK // 2, pair, 0)
    pltpu.make_async_copy(cmp0, out_rows(_NCHUNK - 2), ws0).wait()
    pltpu.make_async_copy(cmp1, out_rows(_NCHUNK - 1), ws1).wait()


def kernel(sentence, W):
    idx = sentence.reshape(_NW, _NCHUNK, _CHUNK)
    Wp = jnp.pad(W, ((0, 0), (0, _DP - _D)))
    out = _emb_gather(idx, Wp)
    return out.reshape(sentence.shape[0], sentence.shape[1], _D)


# speedup vs baseline: 1.3762x; 1.1788x over previous
import functools

import jax
import jax.numpy as jnp
from jax import lax
from jax.experimental import pallas as pl
from jax.experimental.pallas import tpu as pltpu
from jax.experimental.pallas import tpu_sc as plsc

_D = 300
_DP = 320
_B = 4096 * 50
_NC = 2
_NS = 16
_NW = _NC * _NS
_BPW = _B // _NW
_CHUNK = 64
_NCHUNK = _BPW // _CHUNK

_mesh = plsc.VectorSubcoreMesh(core_axis_name="c", subcore_axis_name="s")


@functools.partial(
    pl.kernel,
    mesh=_mesh,
    compiler_params=pltpu.CompilerParams(use_tc_tiling_on_sc=False),
    out_type=jax.ShapeDtypeStruct((_B, _D), jnp.float32),
    scratch_types=[
        pltpu.VMEM((_NCHUNK, _CHUNK), jnp.int32),
        pltpu.VMEM((_CHUNK, _DP), jnp.float32),
        pltpu.VMEM((_CHUNK, _DP), jnp.float32),
        pltpu.VMEM((_CHUNK, _D), jnp.float32),
        pltpu.VMEM((_CHUNK, _D), jnp.float32),
        pltpu.SemaphoreType.DMA,
        pltpu.SemaphoreType.DMA,
        pltpu.SemaphoreType.DMA,
        pltpu.SemaphoreType.DMA,
    ],
)
def _emb_gather(idx_hbm, table_hbm, out_hbm, idx_v,
                pad0, pad1, cmp0, cmp1, gs0, gs1, ws0, ws1):
    wid = lax.axis_index("s") * _NC + lax.axis_index("c")
    base = wid * _BPW
    pltpu.sync_copy(idx_hbm.at[wid], idx_v)
    pltpu.async_copy(table_hbm.at[idx_v.at[0]], pad0, gs0)

    def compact(pad_v, cmp_v):
        @plsc.parallel_loop(0, _CHUNK, unroll=4)
        def _(r):
            for k in range(18):
                cmp_v[r, pl.ds(16 * k, 16)] = pad_v[r, pl.ds(16 * k, 16)]
            cmp_v[r, pl.ds(_D - 16, 16)] = pad_v[r, pl.ds(_D - 16, 16)]

    def out_rows(j):
        return out_hbm.at[pl.ds(base + j * _CHUNK, _CHUNK)]

    def pair(t, carry):
        j0 = 2 * t
        pltpu.async_copy(table_hbm.at[idx_v.at[j0 + 1]], pad1, gs1)
        pltpu.make_async_copy(table_hbm.at[idx_v.at[j0]], pad0, gs0).wait()

        @pl.when(t >= 1)
        def _():
            pltpu.make_async_copy(cmp0, out_rows(j0 - 2), ws0).wait()

        compact(pad0, cmp0)
        pltpu.async_copy(cmp0, out_rows(j0), ws0)

        @pl.when(t < _NCHUNK // 2 - 1)
        def _():
            pltpu.async_copy(table_hbm.at[idx_v.at[j0 + 2]], pad0, gs0)

        pltpu.make_async_copy(table_hbm.at[idx_v.at[j0 + 1]], pad1, gs1).wait()

        @pl.when(t >= 1)
        def _():
            pltpu.make_async_copy(cmp1, out_rows(j0 - 1), ws1).wait()

        compact(pad1, cmp1)
        pltpu.async_copy(cmp1, out_rows(j0 + 1), ws1)
        return carry

    lax.fori_loop(0, _NCHUNK // 2, pair, 0)
    pltpu.make_async_copy(cmp0, out_rows(_NCHUNK - 2), ws0).wait()
    pltpu.make_async_copy(cmp1, out_rows(_NCHUNK - 1), ws1).wait()


def kernel(sentence, W):
    idx = sentence.reshape(_NW, _NCHUNK, _CHUNK)
    Wp = jnp.pad(W, ((0, 0), (0, _DP - _D)))
    out = _emb_gather(idx, Wp)
    return out.reshape(sentence.shape[0], sentence.shape[1], _D)


# speedup vs baseline: 1.5439x; 1.1219x over previous
import functools

import jax
import jax.numpy as jnp
from jax import lax
from jax.experimental import pallas as pl
from jax.experimental.pallas import tpu as pltpu
from jax.experimental.pallas import tpu_sc as plsc

_D = 300
_DP = 320
_B = 4096 * 50
_NC = 2
_NS = 16
_NW = _NC * _NS
_BPW = _B // _NW
_CHUNK = 64
_NCHUNK = _BPW // _CHUNK
_CB = _CHUNK * _D

_mesh = plsc.VectorSubcoreMesh(core_axis_name="c", subcore_axis_name="s")


@functools.partial(
    pl.kernel,
    mesh=_mesh,
    compiler_params=pltpu.CompilerParams(use_tc_tiling_on_sc=False),
    out_type=jax.ShapeDtypeStruct((_B * _D,), jnp.float32),
    scratch_types=[
        pltpu.VMEM((_NCHUNK, _CHUNK), jnp.int32),
        pltpu.VMEM((_CHUNK, _DP), jnp.float32),
        pltpu.VMEM((_CHUNK, _DP), jnp.float32),
        pltpu.VMEM((_CB,), jnp.float32),
        pltpu.VMEM((_CB,), jnp.float32),
        pltpu.SemaphoreType.DMA,
        pltpu.SemaphoreType.DMA,
        pltpu.SemaphoreType.DMA,
        pltpu.SemaphoreType.DMA,
    ],
)
def _emb_gather(idx_hbm, table_hbm, out_hbm, idx_v,
                pad0, pad1, cmp0, cmp1, gs0, gs1, ws0, ws1):
    wid = lax.axis_index("s") * _NC + lax.axis_index("c")
    base = wid * _BPW * _D
    pltpu.sync_copy(idx_hbm.at[wid], idx_v)
    pltpu.async_copy(table_hbm.at[idx_v.at[0]], pad0, gs0)

    def compact(pad_v, cmp_v):
        @plsc.parallel_loop(0, _CHUNK, unroll=4)
        def _(r):
            for k in range(18):
                cmp_v[pl.ds(_D * r + 16 * k, 16)] = pad_v[r, pl.ds(16 * k, 16)]
            cmp_v[pl.ds(_D * r + _D - 16, 16)] = pad_v[r, pl.ds(_D - 16, 16)]

    def out_flat(j):
        return out_hbm.at[pl.ds(base + j * _CB, _CB)]

    def pair(t, carry):
        j0 = 2 * t
        pltpu.async_copy(table_hbm.at[idx_v.at[j0 + 1]], pad1, gs1)
        pltpu.make_async_copy(table_hbm.at[idx_v.at[j0]], pad0, gs0).wait()

        @pl.when(t >= 1)
        def _():
            pltpu.make_async_copy(cmp0, out_flat(j0 - 2), ws0).wait()

        compact(pad0, cmp0)
        pltpu.async_copy(cmp0, out_flat(j0), ws0)

        @pl.when(t < _NCHUNK // 2 - 1)
        def _():
            pltpu.async_copy(table_hbm.at[idx_v.at[j0 + 2]], pad0, gs0)

        pltpu.make_async_copy(table_hbm.at[idx_v.at[j0 + 1]], pad1, gs1).wait()

        @pl.when(t >= 1)
        def _():
            pltpu.make_async_copy(cmp1, out_flat(j0 - 1), ws1).wait()

        compact(pad1, cmp1)
        pltpu.async_copy(cmp1, out_flat(j0 + 1), ws1)
        return carry

    lax.fori_loop(0, _NCHUNK // 2, pair, 0)
    pltpu.make_async_copy(cmp0, out_flat(_NCHUNK - 2), ws0).wait()
    pltpu.make_async_copy(cmp1, out_flat(_NCHUNK - 1), ws1).wait()


def kernel(sentence, W):
    idx = sentence.reshape(_NW, _NCHUNK, _CHUNK)
    Wp = jnp.pad(W, ((0, 0), (0, _DP - _D)))
    out = _emb_gather(idx, Wp)
    return out.reshape(sentence.shape[0], sentence.shape[1], _D)
